# baseline (speedup 1.0000x reference)
import functools

import jax
import jax.numpy as jnp
from jax import lax
from jax.experimental import pallas as pl
from jax.experimental.pallas import tpu as pltpu
from jax.experimental.pallas import tpu_sc as plsc

N = 10000
E = 320000
DIN = 128
DOUT = 128

NC = 2
NS = 16
NW = NC * NS
EP = E // NW
K = 128
NCH = 79
EPP = NCH * K
NPAD = 10240
ROWS_PER_TILE = NPAD // NS
FB = DIN // 16


def _proj_body(a_ref, b_ref, w_ref, o_ref):
    o_ref[...] = lax.dot_general(
        a_ref[...] + b_ref[...], w_ref[...], (((1,), (1,)), ((), ())),
        preferred_element_type=jnp.float32)


def _sc_body(x_hbm, comb_hbm, out_hbm,
             idx0, idx1, rows0, rows1, acc_sh,
             si0, si1, sg0, sg1, ssc):
    c = lax.axis_index("c")
    s = lax.axis_index("s")
    wid = s * NC + c

    idxs = (idx0, idx1)
    rowss = (rows0, rows1)
    sis = (si0, si1)
    sgs = (sg0, sg1)

    def fetch_idx(j, m):
        pltpu.async_copy(comb_hbm.at[wid, j], idxs[m], sis[m])

    def wait_idx(m):
        pltpu.make_async_copy(comb_hbm.at[wid, 0], idxs[m], sis[m]).wait()

    def fire_gather(m):
        pltpu.async_copy(x_hbm.at[idxs[m].at[0, 0]], rowss[m], sgs[m])

    def wait_gather(m):
        pltpu.make_async_copy(x_hbm.at[idxs[m].at[0, 0]], rowss[m],
                              sgs[m]).wait()

    def scale(m):
        rows_v = rowss[m]
        idx_v = idxs[m]

        def grp_body(g, gcarry):
            w16 = lax.bitcast_convert_type(idx_v[2, 0, pl.ds(g * 16, 16)],
                                           jnp.float32)
            base = g * 16
            for e in range(16):
                w = lax.gather(
                    w16, jnp.full((16, 1), e, jnp.int32),
                    lax.GatherDimensionNumbers(
                        offset_dims=(), collapsed_slice_dims=(0,),
                        start_index_map=(0,)),
                    (1,), mode=lax.GatherScatterMode.PROMISE_IN_BOUNDS)
                for f in range(FB):
                    sl = pl.ds(16 * f, 16)
                    rows_v[base + e, sl] = rows_v[base + e, sl] * w
            return gcarry

        lax.fori_loop(0, K // 16, grp_body, 0)

    def scatter(m):
        pltpu.async_copy(rowss[m], acc_sh.at[idxs[m].at[1, 0]], ssc,
                         add=True).wait()

    fetch_idx(0, 0)
    fetch_idx(1, 1)

    zero16 = jnp.zeros((16,), jnp.float32)

    def zrow(i, carry):
        for f in range(FB):
            rows0[i, pl.ds(16 * f, 16)] = zero16
        return carry

    lax.fori_loop(0, K, zrow, 0)
    for r in range(ROWS_PER_TILE // K):
        pltpu.sync_copy(rows0,
                        acc_sh.at[pl.ds(s * ROWS_PER_TILE + r * K, K)])

    wait_idx(0)
    fire_gather(0)
    plsc.subcore_barrier()

    def chunk_step(j, m, fire_next, fetch_mode):
        wait_gather(m)
        scale(m)
        if fire_next:
            wait_idx(1 - m)
            fire_gather(1 - m)
        scatter(m)
        if fetch_mode == "always":
            fetch_idx(j + 2, m)
        elif fetch_mode == "guard":
            @pl.when(j + 2 < NCH)
            def _():
                fetch_idx(j + 2, m)

    def pair_body(p, carry):
        j0 = 2 * p
        chunk_step(j0, 0, True, "always")
        chunk_step(j0 + 1, 1, True, "guard")
        return carry

    lax.fori_loop(0, (NCH - 1) // 2, pair_body, 0)
    chunk_step(NCH - 1, 0, False, "none")

    plsc.subcore_barrier()

    pltpu.sync_copy(acc_sh.at[pl.ds(s * ROWS_PER_TILE, ROWS_PER_TILE)],
                    out_hbm.at[pl.ds(c * NPAD + s * ROWS_PER_TILE,
                                     ROWS_PER_TILE)])


@functools.cache
def _sc_gather_scale_scatter():
    return pl.kernel(
        _sc_body,
        out_type=jax.ShapeDtypeStruct((NC * NPAD, DIN), jnp.float32),
        mesh=plsc.VectorSubcoreMesh(core_axis_name="c", subcore_axis_name="s",
                                    num_cores=NC, num_subcores=NS),
        scratch_types=[
            pltpu.VMEM((3, 1, K), jnp.int32),
            pltpu.VMEM((3, 1, K), jnp.int32),
            pltpu.VMEM((K, DIN), jnp.float32),
            pltpu.VMEM((K, DIN), jnp.float32),
            pltpu.VMEM_SHARED((NPAD, DIN), jnp.float32),
            pltpu.SemaphoreType.DMA,
            pltpu.SemaphoreType.DMA,
            pltpu.SemaphoreType.DMA,
            pltpu.SemaphoreType.DMA,
            pltpu.SemaphoreType.DMA,
        ],
    )


@jax.jit
def kernel(x, edge_index, edge_weight, W):
    pad = EPP - EP
    wid_col = jnp.arange(NW, dtype=jnp.int32)[:, None]
    src = jnp.pad(edge_index[1].reshape(NW, EP), ((0, 0), (0, pad)))
    dst = jnp.concatenate(
        [edge_index[0].reshape(NW, EP),
         jnp.broadcast_to(N + wid_col, (NW, pad))], axis=1)
    ewb = jnp.pad(
        lax.bitcast_convert_type(edge_weight, jnp.int32).reshape(NW, EP),
        ((0, 0), (0, pad)))
    comb = jnp.concatenate(
        [a.reshape(NW, NCH, 1, 1, K) for a in (src, dst, ewb)],
        axis=2)

    partial = _sc_gather_scale_scatter()(x, comb)

    spec = pl.BlockSpec((N // 10, DIN), lambda i: (i, 0))
    out = pl.pallas_call(
        _proj_body,
        grid=(10,),
        in_specs=[spec, spec, pl.BlockSpec((DOUT, DIN), lambda i: (0, 0))],
        out_specs=pl.BlockSpec((N // 10, DOUT), lambda i: (i, 0)),
        out_shape=jax.ShapeDtypeStruct((N, DOUT), jnp.float32),
    )(partial[:N], partial[NPAD:NPAD + N], W)
    return out


# speedup vs baseline: 1.3208x; 1.3208x over previous
import functools

import jax
import jax.numpy as jnp
from jax import lax
from jax.experimental import pallas as pl
from jax.experimental.pallas import tpu as pltpu
from jax.experimental.pallas import tpu_sc as plsc

N = 10000
E = 320000
DIN = 128
DOUT = 128

NC = 2
NS = 16
NW = NC * NS
EP = E // NW
K = 80
NCH = EP // K
EPP = NCH * K
NPAD = 10240
ROWS_PER_TILE = NPAD // NS
FB = DIN // 16


def _proj_body(a_ref, b_ref, w_ref, o_ref):
    o_ref[...] = lax.dot_general(
        a_ref[...] + b_ref[...], w_ref[...], (((1,), (1,)), ((), ())),
        preferred_element_type=jnp.float32)


def _sc_body(x_hbm, comb_hbm, out_hbm,
             idx0, idx1, rows0, rows1, acc_sh,
             si0, si1, sg0, sg1, ssc):
    c = lax.axis_index("c")
    s = lax.axis_index("s")
    wid = s * NC + c

    idxs = (idx0, idx1)
    rowss = (rows0, rows1)
    sis = (si0, si1)
    sgs = (sg0, sg1)

    def fetch_idx(j, m):
        pltpu.async_copy(comb_hbm.at[wid, j], idxs[m], sis[m])

    def wait_idx(m):
        pltpu.make_async_copy(comb_hbm.at[wid, 0], idxs[m], sis[m]).wait()

    def fire_gather(m):
        pltpu.async_copy(x_hbm.at[idxs[m].at[0, 0]], rowss[m], sgs[m])

    def wait_gather(m):
        pltpu.make_async_copy(x_hbm.at[idxs[m].at[0, 0]], rowss[m],
                              sgs[m]).wait()

    def scale(m):
        rows_v = rowss[m]
        idx_v = idxs[m]

        def grp_body(g, gcarry):
            w16 = lax.bitcast_convert_type(idx_v[2, 0, pl.ds(g * 16, 16)],
                                           jnp.float32)
            base = g * 16
            for e in range(16):
                w = lax.gather(
                    w16, jnp.full((16, 1), e, jnp.int32),
                    lax.GatherDimensionNumbers(
                        offset_dims=(), collapsed_slice_dims=(0,),
                        start_index_map=(0,)),
                    (1,), mode=lax.GatherScatterMode.PROMISE_IN_BOUNDS)
                for f in range(FB):
                    sl = pl.ds(16 * f, 16)
                    rows_v[base + e, sl] = rows_v[base + e, sl] * w
            return gcarry

        lax.fori_loop(0, K // 16, grp_body, 0)

    def scatter(m):
        pltpu.async_copy(rowss[m], acc_sh.at[idxs[m].at[1, 0]], ssc,
                         add=True).wait()

    fetch_idx(0, 0)
    fetch_idx(1, 1)

    zero16 = jnp.zeros((16,), jnp.float32)

    def zrow(i, carry):
        for f in range(FB):
            rows0[i, pl.ds(16 * f, 16)] = zero16
        return carry

    lax.fori_loop(0, K, zrow, 0)
    for r in range(ROWS_PER_TILE // K):
        pltpu.sync_copy(rows0,
                        acc_sh.at[pl.ds(s * ROWS_PER_TILE + r * K, K)])

    wait_idx(0)
    fire_gather(0)
    plsc.subcore_barrier()

    def chunk_step(j, m, fire_next, fetch_mode):
        wait_gather(m)
        scale(m)
        if fire_next:
            wait_idx(1 - m)
            fire_gather(1 - m)
        scatter(m)
        if fetch_mode == "always":
            fetch_idx(j + 2, m)
        elif fetch_mode == "guard":
            @pl.when(j + 2 < NCH)
            def _():
                fetch_idx(j + 2, m)

    def pair_body(p, carry):
        j0 = 2 * p
        chunk_step(j0, 0, True, "always")
        chunk_step(j0 + 1, 1, True, "guard")
        return carry

    lax.fori_loop(0, (NCH - 1) // 2, pair_body, 0)
    chunk_step(NCH - 1, 0, False, "none")

    plsc.subcore_barrier()

    pltpu.sync_copy(acc_sh.at[pl.ds(s * ROWS_PER_TILE, ROWS_PER_TILE)],
                    out_hbm.at[pl.ds(c * NPAD + s * ROWS_PER_TILE,
                                     ROWS_PER_TILE)])


@functools.cache
def _sc_gather_scale_scatter():
    return pl.kernel(
        _sc_body,
        out_type=jax.ShapeDtypeStruct((NC * NPAD, DIN), jnp.float32),
        mesh=plsc.VectorSubcoreMesh(core_axis_name="c", subcore_axis_name="s",
                                    num_cores=NC, num_subcores=NS),
        scratch_types=[
            pltpu.VMEM((3, 1, K), jnp.int32),
            pltpu.VMEM((3, 1, K), jnp.int32),
            pltpu.VMEM((K, DIN), jnp.float32),
            pltpu.VMEM((K, DIN), jnp.float32),
            pltpu.VMEM_SHARED((NPAD, DIN), jnp.float32),
            pltpu.SemaphoreType.DMA,
            pltpu.SemaphoreType.DMA,
            pltpu.SemaphoreType.DMA,
            pltpu.SemaphoreType.DMA,
            pltpu.SemaphoreType.DMA,
        ],
    )


@jax.jit
def kernel(x, edge_index, edge_weight, W):
    src = edge_index[1].reshape(NW, EP)
    dst = edge_index[0].reshape(NW, EP)
    ewb = lax.bitcast_convert_type(edge_weight, jnp.int32).reshape(NW, EP)
    comb = jnp.concatenate(
        [a.reshape(NW, NCH, 1, 1, K) for a in (src, dst, ewb)],
        axis=2)

    partial = _sc_gather_scale_scatter()(x, comb)

    spec = pl.BlockSpec((N // 10, DIN), lambda i: (i, 0))
    out = pl.pallas_call(
        _proj_body,
        grid=(10,),
        in_specs=[spec, spec, pl.BlockSpec((DOUT, DIN), lambda i: (0, 0))],
        out_specs=pl.BlockSpec((N // 10, DOUT), lambda i: (i, 0)),
        out_shape=jax.ShapeDtypeStruct((N, DOUT), jnp.float32),
    )(partial[:N], partial[NPAD:NPAD + N], W)
    return out


# speedup vs baseline: 1.6316x; 1.2353x over previous
import functools

import jax
import jax.numpy as jnp
from jax import lax
from jax.experimental import pallas as pl
from jax.experimental.pallas import tpu as pltpu
from jax.experimental.pallas import tpu_sc as plsc

N = 10000
E = 320000
DIN = 128
DOUT = 128

NC = 2
NS = 16
NW = NC * NS
EP = E // NW
K = 80
NCH = EP // K
EPP = NCH * K
NPAD = 10240
ROWS_PER_TILE = NPAD // NS
FB = DIN // 16


def _proj_body(a_ref, b_ref, w_ref, o_ref):
    o_ref[...] = lax.dot_general(
        a_ref[...] + b_ref[...], w_ref[...], (((1,), (1,)), ((), ())),
        preferred_element_type=jnp.float32)


def _sc_body(x_hbm, comb_hbm, out_hbm,
             idx0, idx1, idx2, rows0, rows1, rows2, acc_sh,
             si0, si1, si2, sg0, sg1, sg2, ssc):
    c = lax.axis_index("c")
    s = lax.axis_index("s")
    wid = s * NC + c

    idxs = (idx0, idx1, idx2)
    rowss = (rows0, rows1, rows2)
    sis = (si0, si1, si2)
    sgs = (sg0, sg1, sg2)

    def fetch_idx(j, m):
        pltpu.async_copy(comb_hbm.at[wid, j], idxs[m], sis[m])

    def wait_idx(m):
        pltpu.make_async_copy(comb_hbm.at[wid, 0], idxs[m], sis[m]).wait()

    def fire_gather(m):
        pltpu.async_copy(x_hbm.at[idxs[m].at[0, 0]], rowss[m], sgs[m])

    def wait_gather(m):
        pltpu.make_async_copy(x_hbm.at[idxs[m].at[0, 0]], rowss[m],
                              sgs[m]).wait()

    def scale(m):
        rows_v = rowss[m]
        idx_v = idxs[m]

        def grp_body(g, gcarry):
            w16 = lax.bitcast_convert_type(idx_v[2, 0, pl.ds(g * 16, 16)],
                                           jnp.float32)
            base = g * 16
            for e in range(16):
                w = lax.gather(
                    w16, jnp.full((16, 1), e, jnp.int32),
                    lax.GatherDimensionNumbers(
                        offset_dims=(), collapsed_slice_dims=(0,),
                        start_index_map=(0,)),
                    (1,), mode=lax.GatherScatterMode.PROMISE_IN_BOUNDS)
                for f in range(FB):
                    sl = pl.ds(16 * f, 16)
                    rows_v[base + e, sl] = rows_v[base + e, sl] * w
            return gcarry

        lax.fori_loop(0, K // 16, grp_body, 0)

    def scatter(m):
        pltpu.async_copy(rowss[m], acc_sh.at[idxs[m].at[1, 0]], ssc,
                         add=True).wait()

    fetch_idx(0, 0)
    fetch_idx(1, 1)
    fetch_idx(2, 2)

    zero16 = jnp.zeros((16,), jnp.float32)

    def zrow(i, carry):
        for f in range(FB):
            rows0[i, pl.ds(16 * f, 16)] = zero16
        return carry

    lax.fori_loop(0, K, zrow, 0)
    for r in range(ROWS_PER_TILE // K):
        pltpu.sync_copy(rows0,
                        acc_sh.at[pl.ds(s * ROWS_PER_TILE + r * K, K)])

    wait_idx(0)
    fire_gather(0)
    wait_idx(1)
    fire_gather(1)
    plsc.subcore_barrier()

    def chunk_step(j, m, fire_next, fetch_mode):
        wait_gather(m)
        scale(m)
        m2 = (m + 2) % 3
        if fire_next:
            wait_idx(m2)
            fire_gather(m2)
        scatter(m)
        if fetch_mode == "always":
            fetch_idx(j + 3, m)
        elif fetch_mode == "guard":
            @pl.when(j + 3 < NCH)
            def _():
                fetch_idx(j + 3, m)

    def triple_body(p, carry):
        j0 = 3 * p
        chunk_step(j0, 0, True, "always")
        chunk_step(j0 + 1, 1, True, "always")
        chunk_step(j0 + 2, 2, True, "guard")
        return carry

    lax.fori_loop(0, (NCH - 2) // 3, triple_body, 0)
    chunk_step(NCH - 2, 0, False, "none")
    chunk_step(NCH - 1, 1, False, "none")

    plsc.subcore_barrier()

    pltpu.sync_copy(acc_sh.at[pl.ds(s * ROWS_PER_TILE, ROWS_PER_TILE)],
                    out_hbm.at[pl.ds(c * NPAD + s * ROWS_PER_TILE,
                                     ROWS_PER_TILE)])


@functools.cache
def _sc_gather_scale_scatter():
    return pl.kernel(
        _sc_body,
        out_type=jax.ShapeDtypeStruct((NC * NPAD, DIN), jnp.float32),
        mesh=plsc.VectorSubcoreMesh(core_axis_name="c", subcore_axis_name="s",
                                    num_cores=NC, num_subcores=NS),
        scratch_types=[
            pltpu.VMEM((3, 1, K), jnp.int32),
            pltpu.VMEM((3, 1, K), jnp.int32),
            pltpu.VMEM((3, 1, K), jnp.int32),
            pltpu.VMEM((K, DIN), jnp.float32),
            pltpu.VMEM((K, DIN), jnp.float32),
            pltpu.VMEM((K, DIN), jnp.float32),
            pltpu.VMEM_SHARED((NPAD, DIN), jnp.float32),
            pltpu.SemaphoreType.DMA,
            pltpu.SemaphoreType.DMA,
            pltpu.SemaphoreType.DMA,
            pltpu.SemaphoreType.DMA,
            pltpu.SemaphoreType.DMA,
            pltpu.SemaphoreType.DMA,
            pltpu.SemaphoreType.DMA,
        ],
    )


@jax.jit
def kernel(x, edge_index, edge_weight, W):
    src = edge_index[1].reshape(NW, EP)
    dst = edge_index[0].reshape(NW, EP)
    ewb = lax.bitcast_convert_type(edge_weight, jnp.int32).reshape(NW, EP)
    comb = jnp.concatenate(
        [a.reshape(NW, NCH, 1, 1, K) for a in (src, dst, ewb)],
        axis=2)

    partial = _sc_gather_scale_scatter()(x, comb)

    spec = pl.BlockSpec((N // 10, DIN), lambda i: (i, 0))
    out = pl.pallas_call(
        _proj_body,
        grid=(10,),
        in_specs=[spec, spec, pl.BlockSpec((DOUT, DIN), lambda i: (0, 0))],
        out_specs=pl.BlockSpec((N // 10, DOUT), lambda i: (i, 0)),
        out_shape=jax.ShapeDtypeStruct((N, DOUT), jnp.float32),
    )(partial[:N], partial[NPAD:NPAD + N], W)
    return out


# speedup vs baseline: 1.8670x; 1.1442x over previous
import functools

import jax
import jax.numpy as jnp
from jax import lax
from jax.experimental import pallas as pl
from jax.experimental.pallas import tpu as pltpu
from jax.experimental.pallas import tpu_sc as plsc

N = 10000
E = 320000
DIN = 128
DOUT = 128

NC = 2
NS = 16
NW = NC * NS
EP = E // NW
K = 80
NCH = EP // K
EPP = NCH * K
NPAD = 10240
ROWS_PER_TILE = NPAD // NS
FB = DIN // 16


def _proj_body(a_ref, b_ref, w_ref, o_ref):
    o_ref[...] = lax.dot_general(
        a_ref[...] + b_ref[...], w_ref[...], (((1,), (1,)), ((), ())),
        preferred_element_type=jnp.float32)


def _sc_body(x_hbm, comb_hbm, out_hbm,
             idx0, idx1, idx2, idx3, idx4, idx5,
             rows0, rows1, rows2, acc_sh,
             si0, si1, si2, si3, si4, si5, sg0, sg1, sg2, ssc):
    c = lax.axis_index("c")
    s = lax.axis_index("s")
    wid = s * NC + c

    idxs = (idx0, idx1, idx2, idx3, idx4, idx5)
    rowss = (rows0, rows1, rows2)
    sis = (si0, si1, si2, si3, si4, si5)
    sgs = (sg0, sg1, sg2)

    def fetch_idx(j, q):
        pltpu.async_copy(comb_hbm.at[wid, j], idxs[q], sis[q])

    def wait_idx(q):
        pltpu.make_async_copy(comb_hbm.at[wid, 0], idxs[q], sis[q]).wait()

    def fire_gather(m, q):
        pltpu.async_copy(x_hbm.at[idxs[q].at[0, 0]], rowss[m], sgs[m])

    def wait_gather(m, q):
        pltpu.make_async_copy(x_hbm.at[idxs[q].at[0, 0]], rowss[m],
                              sgs[m]).wait()

    def scale(m, q):
        rows_v = rowss[m]
        idx_v = idxs[q]

        def grp_body(g, gcarry):
            w16 = lax.bitcast_convert_type(idx_v[2, 0, pl.ds(g * 16, 16)],
                                           jnp.float32)
            base = g * 16
            for e in range(16):
                w = lax.gather(
                    w16, jnp.full((16, 1), e, jnp.int32),
                    lax.GatherDimensionNumbers(
                        offset_dims=(), collapsed_slice_dims=(0,),
                        start_index_map=(0,)),
                    (1,), mode=lax.GatherScatterMode.PROMISE_IN_BOUNDS)
                for f in range(FB):
                    sl = pl.ds(16 * f, 16)
                    rows_v[base + e, sl] = rows_v[base + e, sl] * w
            return gcarry

        lax.fori_loop(0, K // 16, grp_body, 0)

    def fire_scatter(m, q):
        pltpu.async_copy(rowss[m], acc_sh.at[idxs[q].at[1, 0]], ssc,
                         add=True)

    def wait_scatter(m):
        pltpu.make_async_copy(x_hbm.at[pl.ds(0, K)], rowss[m], ssc).wait()

    fetch_idx(0, 0)
    fetch_idx(1, 1)
    fetch_idx(2, 2)
    fetch_idx(3, 3)

    zero16 = jnp.zeros((16,), jnp.float32)

    def zrow(i, carry):
        for f in range(FB):
            rows0[i, pl.ds(16 * f, 16)] = zero16
        return carry

    lax.fori_loop(0, K, zrow, 0)
    for r in range(ROWS_PER_TILE // K):
        pltpu.sync_copy(rows0,
                        acc_sh.at[pl.ds(s * ROWS_PER_TILE + r * K, K)])

    wait_idx(0)
    fire_gather(0, 0)
    wait_idx(1)
    fire_gather(1, 1)
    plsc.subcore_barrier()

    def chunk_step(j, m, q, wait_sc, fire_next, do_fetch):
        wait_gather(m, q)
        scale(m, q)
        if wait_sc == "always":
            wait_scatter((m + 2) % 3)
        elif wait_sc == "guard":
            @pl.when(j > 0)
            def _():
                wait_scatter((m + 2) % 3)
        if fire_next:
            wait_idx((q + 2) % 6)
            fire_gather((m + 2) % 3, (q + 2) % 6)
        fire_scatter(m, q)
        if do_fetch:
            fetch_idx(j + 4, (q + 4) % 6)

    def six_body(p, carry):
        j0 = 6 * p
        chunk_step(j0, 0, 0, "guard", True, True)
        chunk_step(j0 + 1, 1, 1, "always", True, True)
        chunk_step(j0 + 2, 2, 2, "always", True, True)
        chunk_step(j0 + 3, 0, 3, "always", True, True)
        chunk_step(j0 + 4, 1, 4, "always", True, True)
        chunk_step(j0 + 5, 2, 5, "always", True, True)
        return carry

    lax.fori_loop(0, NCH // 6, six_body, 0)
    chunk_step(120, 0, 0, "always", True, True)
    chunk_step(121, 1, 1, "always", True, False)
    chunk_step(122, 2, 2, "always", True, False)
    chunk_step(123, 0, 3, "always", False, False)
    chunk_step(124, 1, 4, "always", False, False)
    wait_scatter(1)

    plsc.subcore_barrier()

    pltpu.sync_copy(acc_sh.at[pl.ds(s * ROWS_PER_TILE, ROWS_PER_TILE)],
                    out_hbm.at[pl.ds(c * NPAD + s * ROWS_PER_TILE,
                                     ROWS_PER_TILE)])


@functools.cache
def _sc_gather_scale_scatter():
    return pl.kernel(
        _sc_body,
        out_type=jax.ShapeDtypeStruct((NC * NPAD, DIN), jnp.float32),
        mesh=plsc.VectorSubcoreMesh(core_axis_name="c", subcore_axis_name="s",
                                    num_cores=NC, num_subcores=NS),
        scratch_types=[
            pltpu.VMEM((3, 1, K), jnp.int32),
            pltpu.VMEM((3, 1, K), jnp.int32),
            pltpu.VMEM((3, 1, K), jnp.int32),
            pltpu.VMEM((3, 1, K), jnp.int32),
            pltpu.VMEM((3, 1, K), jnp.int32),
            pltpu.VMEM((3, 1, K), jnp.int32),
            pltpu.VMEM((K, DIN), jnp.float32),
            pltpu.VMEM((K, DIN), jnp.float32),
            pltpu.VMEM((K, DIN), jnp.float32),
            pltpu.VMEM_SHARED((NPAD, DIN), jnp.float32),
            pltpu.SemaphoreType.DMA,
            pltpu.SemaphoreType.DMA,
            pltpu.SemaphoreType.DMA,
            pltpu.SemaphoreType.DMA,
            pltpu.SemaphoreType.DMA,
            pltpu.SemaphoreType.DMA,
            pltpu.SemaphoreType.DMA,
            pltpu.SemaphoreType.DMA,
            pltpu.SemaphoreType.DMA,
            pltpu.SemaphoreType.DMA,
        ],
    )


@jax.jit
def kernel(x, edge_index, edge_weight, W):
    src = edge_index[1].reshape(NW, EP)
    dst = edge_index[0].reshape(NW, EP)
    ewb = lax.bitcast_convert_type(edge_weight, jnp.int32).reshape(NW, EP)
    comb = jnp.concatenate(
        [a.reshape(NW, NCH, 1, 1, K) for a in (src, dst, ewb)],
        axis=2)

    partial = _sc_gather_scale_scatter()(x, comb)

    spec = pl.BlockSpec((N // 10, DIN), lambda i: (i, 0))
    out = pl.pallas_call(
        _proj_body,
        grid=(10,),
        in_specs=[spec, spec, pl.BlockSpec((DOUT, DIN), lambda i: (0, 0))],
        out_specs=pl.BlockSpec((N // 10, DOUT), lambda i: (i, 0)),
        out_shape=jax.ShapeDtypeStruct((N, DOUT), jnp.float32),
    )(partial[:N], partial[NPAD:NPAD + N], W)
    return out


# speedup vs baseline: 1.9599x; 1.0498x over previous
import functools

import jax
import jax.numpy as jnp
from jax import lax
from jax.experimental import pallas as pl
from jax.experimental.pallas import tpu as pltpu
from jax.experimental.pallas import tpu_sc as plsc

N = 10000
E = 320000
DIN = 128
DOUT = 128

NC = 2
NS = 16
NW = NC * NS
EP = E // NW
K = 80
NCH = EP // K
EPP = NCH * K
NPAD = 10240
ROWS_PER_TILE = NPAD // NS
FB = DIN // 16


def _proj_body(a_ref, b_ref, w_ref, o_ref):
    o_ref[...] = lax.dot_general(
        a_ref[...] + b_ref[...], w_ref[...], (((1,), (1,)), ((), ())),
        preferred_element_type=jnp.float32)


def _sc_body(x_hbm, src_hbm, dst_hbm, ewb_hbm, out_hbm,
             idx0, idx1, idx2, idx3, idx4, idx5,
             rows0, rows1, rows2, acc_sh,
             si0, si1, si2, si3, si4, si5, sg0, sg1, sg2, ssc):
    c = lax.axis_index("c")
    s = lax.axis_index("s")
    wid = s * NC + c

    idxs = (idx0, idx1, idx2, idx3, idx4, idx5)
    rowss = (rows0, rows1, rows2)
    sis = (si0, si1, si2, si3, si4, si5)
    sgs = (sg0, sg1, sg2)

    def fetch_idx(j, q):
        pltpu.async_copy(src_hbm.at[wid, j], idxs[q].at[0], sis[q])
        pltpu.async_copy(dst_hbm.at[wid, j], idxs[q].at[1], sis[q])
        pltpu.async_copy(ewb_hbm.at[wid, j], idxs[q].at[2], sis[q])

    def wait_idx(q):
        pltpu.make_async_copy(src_hbm.at[wid, 0], idxs[q].at[0], sis[q]).wait()
        pltpu.make_async_copy(dst_hbm.at[wid, 0], idxs[q].at[1], sis[q]).wait()
        pltpu.make_async_copy(ewb_hbm.at[wid, 0], idxs[q].at[2], sis[q]).wait()

    def fire_gather(m, q):
        pltpu.async_copy(x_hbm.at[idxs[q].at[0, 0]], rowss[m], sgs[m])

    def wait_gather(m, q):
        pltpu.make_async_copy(x_hbm.at[idxs[q].at[0, 0]], rowss[m],
                              sgs[m]).wait()

    def scale(m, q):
        rows_v = rowss[m]
        idx_v = idxs[q]

        def grp_body(g, gcarry):
            w16 = lax.bitcast_convert_type(idx_v[2, 0, pl.ds(g * 16, 16)],
                                           jnp.float32)
            base = g * 16
            for e in range(16):
                w = lax.gather(
                    w16, jnp.full((16, 1), e, jnp.int32),
                    lax.GatherDimensionNumbers(
                        offset_dims=(), collapsed_slice_dims=(0,),
                        start_index_map=(0,)),
                    (1,), mode=lax.GatherScatterMode.PROMISE_IN_BOUNDS)
                for f in range(FB):
                    sl = pl.ds(16 * f, 16)
                    rows_v[base + e, sl] = rows_v[base + e, sl] * w
            return gcarry

        lax.fori_loop(0, K // 16, grp_body, 0)

    def fire_scatter(m, q):
        pltpu.async_copy(rowss[m], acc_sh.at[idxs[q].at[1, 0]], ssc,
                         add=True)

    def wait_scatter(m):
        pltpu.make_async_copy(x_hbm.at[pl.ds(0, K)], rowss[m], ssc).wait()

    fetch_idx(0, 0)
    fetch_idx(1, 1)
    fetch_idx(2, 2)
    fetch_idx(3, 3)

    wait_idx(0)
    fire_gather(0, 0)
    wait_idx(1)
    fire_gather(1, 1)

    zero16 = jnp.zeros((16,), jnp.float32)

    def zrow(i, carry):
        for f in range(FB):
            rows2[i, pl.ds(16 * f, 16)] = zero16
        return carry

    lax.fori_loop(0, K, zrow, 0)
    for r in range(ROWS_PER_TILE // K):
        pltpu.sync_copy(rows2,
                        acc_sh.at[pl.ds(s * ROWS_PER_TILE + r * K, K)])
    plsc.subcore_barrier()

    def chunk_step(j, m, q, wait_sc, fire_next, do_fetch):
        wait_gather(m, q)
        scale(m, q)
        if wait_sc == "always":
            wait_scatter((m + 2) % 3)
        elif wait_sc == "guard":
            @pl.when(j > 0)
            def _():
                wait_scatter((m + 2) % 3)
        if fire_next:
            wait_idx((q + 2) % 6)
            fire_gather((m + 2) % 3, (q + 2) % 6)
        fire_scatter(m, q)
        if do_fetch:
            fetch_idx(j + 4, (q + 4) % 6)

    def six_body(p, carry):
        j0 = 6 * p
        chunk_step(j0, 0, 0, "guard", True, True)
        chunk_step(j0 + 1, 1, 1, "always", True, True)
        chunk_step(j0 + 2, 2, 2, "always", True, True)
        chunk_step(j0 + 3, 0, 3, "always", True, True)
        chunk_step(j0 + 4, 1, 4, "always", True, True)
        chunk_step(j0 + 5, 2, 5, "always", True, True)
        return carry

    lax.fori_loop(0, NCH // 6, six_body, 0)
    chunk_step(120, 0, 0, "always", True, True)
    chunk_step(121, 1, 1, "always", True, False)
    chunk_step(122, 2, 2, "always", True, False)
    chunk_step(123, 0, 3, "always", False, False)
    chunk_step(124, 1, 4, "always", False, False)
    wait_scatter(1)

    plsc.subcore_barrier()

    pltpu.sync_copy(acc_sh.at[pl.ds(s * ROWS_PER_TILE, ROWS_PER_TILE)],
                    out_hbm.at[pl.ds(c * NPAD + s * ROWS_PER_TILE,
                                     ROWS_PER_TILE)])


@functools.cache
def _sc_gather_scale_scatter():
    return pl.kernel(
        _sc_body,
        out_type=jax.ShapeDtypeStruct((NC * NPAD, DIN), jnp.float32),
        mesh=plsc.VectorSubcoreMesh(core_axis_name="c", subcore_axis_name="s",
                                    num_cores=NC, num_subcores=NS),
        scratch_types=[
            pltpu.VMEM((3, 1, K), jnp.int32),
            pltpu.VMEM((3, 1, K), jnp.int32),
            pltpu.VMEM((3, 1, K), jnp.int32),
            pltpu.VMEM((3, 1, K), jnp.int32),
            pltpu.VMEM((3, 1, K), jnp.int32),
            pltpu.VMEM((3, 1, K), jnp.int32),
            pltpu.VMEM((K, DIN), jnp.float32),
            pltpu.VMEM((K, DIN), jnp.float32),
            pltpu.VMEM((K, DIN), jnp.float32),
            pltpu.VMEM_SHARED((NPAD, DIN), jnp.float32),
            pltpu.SemaphoreType.DMA,
            pltpu.SemaphoreType.DMA,
            pltpu.SemaphoreType.DMA,
            pltpu.SemaphoreType.DMA,
            pltpu.SemaphoreType.DMA,
            pltpu.SemaphoreType.DMA,
            pltpu.SemaphoreType.DMA,
            pltpu.SemaphoreType.DMA,
            pltpu.SemaphoreType.DMA,
            pltpu.SemaphoreType.DMA,
        ],
    )


@jax.jit
def kernel(x, edge_index, edge_weight, W):
    src = edge_index[1].reshape(NW, NCH, 1, K)
    dst = edge_index[0].reshape(NW, NCH, 1, K)
    ewb = lax.bitcast_convert_type(edge_weight, jnp.int32).reshape(
        NW, NCH, 1, K)

    partial = _sc_gather_scale_scatter()(x, src, dst, ewb)

    spec = pl.BlockSpec((N // 10, DIN), lambda i: (i, 0))
    out = pl.pallas_call(
        _proj_body,
        grid=(10,),
        in_specs=[spec, spec, pl.BlockSpec((DOUT, DIN), lambda i: (0, 0))],
        out_specs=pl.BlockSpec((N // 10, DOUT), lambda i: (i, 0)),
        out_shape=jax.ShapeDtypeStruct((N, DOUT), jnp.float32),
    )(partial[:N], partial[NPAD:NPAD + N], W)
    return out


# speedup vs baseline: 2.0305x; 1.0360x over previous
import functools

import jax
import jax.numpy as jnp
from jax import lax
from jax.experimental import pallas as pl
from jax.experimental.pallas import tpu as pltpu
from jax.experimental.pallas import tpu_sc as plsc

N = 10000
E = 320000
DIN = 128
DOUT = 128

NC = 2
NS = 16
NW = NC * NS
EP = E // NW
K = 80
NCH = EP // K
EPP = NCH * K
NPAD = 10240
ROWS_PER_TILE = NPAD // NS
FB = DIN // 16


def _proj_body(a_ref, b_ref, w_ref, o_ref):
    o_ref[...] = lax.dot_general(
        a_ref[...] + b_ref[...], w_ref[...], (((1,), (1,)), ((), ())),
        preferred_element_type=jnp.float32)


def _sc_body(x_hbm, src_hbm, dst_hbm, ewb_hbm, out_hbm,
             idx0, idx1, idx2, idx3, idx4, idx5,
             rows0, rows1, rows2, acc_sh,
             si0, si1, si2, si3, si4, si5, sg0, sg1, sg2, ssc):
    c = lax.axis_index("c")
    s = lax.axis_index("s")
    wid = s * NC + c

    idxs = (idx0, idx1, idx2, idx3, idx4, idx5)
    rowss = (rows0, rows1, rows2)
    sis = (si0, si1, si2, si3, si4, si5)
    sgs = (sg0, sg1, sg2)

    def fetch_idx(j, q):
        pltpu.async_copy(src_hbm.at[wid, j], idxs[q].at[0], sis[q])
        pltpu.async_copy(dst_hbm.at[wid, j], idxs[q].at[1], sis[q])
        pltpu.async_copy(ewb_hbm.at[wid, j], idxs[q].at[2], sis[q])

    def wait_idx(q):
        pltpu.make_async_copy(src_hbm.at[wid, 0], idxs[q].at[0], sis[q]).wait()
        pltpu.make_async_copy(dst_hbm.at[wid, 0], idxs[q].at[1], sis[q]).wait()
        pltpu.make_async_copy(ewb_hbm.at[wid, 0], idxs[q].at[2], sis[q]).wait()

    def fire_gather(m, q):
        pltpu.async_copy(x_hbm.at[idxs[q].at[0, 0]], rowss[m], sgs[m])

    def wait_gather(m, q):
        pltpu.make_async_copy(x_hbm.at[idxs[q].at[0, 0]], rowss[m],
                              sgs[m]).wait()

    def scale(m, q):
        rows_v = rowss[m]
        idx_v = idxs[q]

        def grp_body(g, gcarry):
            w16 = lax.bitcast_convert_type(idx_v[2, 0, pl.ds(g * 16, 16)],
                                           jnp.float32)
            base = g * 16
            for e in range(16):
                w = lax.gather(
                    w16, jnp.full((16, 1), e, jnp.int32),
                    lax.GatherDimensionNumbers(
                        offset_dims=(), collapsed_slice_dims=(0,),
                        start_index_map=(0,)),
                    (1,), mode=lax.GatherScatterMode.PROMISE_IN_BOUNDS)
                for f in range(FB):
                    sl = pl.ds(16 * f, 16)
                    rows_v[base + e, sl] = rows_v[base + e, sl] * w
            return gcarry

        lax.fori_loop(0, K // 16, grp_body, 0)

    def fire_scatter(m, q):
        pltpu.async_copy(rowss[m], acc_sh.at[idxs[q].at[1, 0]], ssc,
                         add=True)

    def wait_scatter(m):
        pltpu.make_async_copy(x_hbm.at[pl.ds(0, K)], rowss[m], ssc).wait()

    fetch_idx(0, 0)
    fetch_idx(1, 1)
    fetch_idx(2, 2)
    fetch_idx(3, 3)

    wait_idx(0)
    fire_gather(0, 0)
    wait_idx(1)
    fire_gather(1, 1)

    zero16 = jnp.zeros((16,), jnp.float32)

    def zrow(i, carry):
        for f in range(FB):
            rows2[i, pl.ds(16 * f, 16)] = zero16
        return carry

    lax.fori_loop(0, K, zrow, 0)
    for r in range(ROWS_PER_TILE // K):
        pltpu.sync_copy(rows2,
                        acc_sh.at[pl.ds(s * ROWS_PER_TILE + r * K, K)])
    plsc.subcore_barrier()

    def chunk_step(j, m, q, wait_sc, fire_next, do_fetch):
        wait_gather(m, q)
        scale(m, q)
        if wait_sc == "always":
            wait_scatter((m + 2) % 3)
        elif wait_sc == "guard":
            @pl.when(j > 0)
            def _():
                wait_scatter((m + 2) % 3)
        if fire_next:
            wait_idx((q + 2) % 6)
            fire_gather((m + 2) % 3, (q + 2) % 6)
        fire_scatter(m, q)
        if do_fetch:
            fetch_idx(j + 4, (q + 4) % 6)

    def six_body(p, carry):
        j0 = 6 * p
        chunk_step(j0, 0, 0, "guard", True, True)
        chunk_step(j0 + 1, 1, 1, "always", True, True)
        chunk_step(j0 + 2, 2, 2, "always", True, True)
        chunk_step(j0 + 3, 0, 3, "always", True, True)
        chunk_step(j0 + 4, 1, 4, "always", True, True)
        chunk_step(j0 + 5, 2, 5, "always", True, True)
        return carry

    lax.fori_loop(0, NCH // 6, six_body, 0)
    chunk_step(120, 0, 0, "always", True, True)
    chunk_step(121, 1, 1, "always", True, False)
    chunk_step(122, 2, 2, "always", True, False)
    chunk_step(123, 0, 3, "always", False, False)
    chunk_step(124, 1, 4, "always", False, False)
    wait_scatter(1)

    plsc.subcore_barrier()

    tail_rows = N - (NS - 1) * ROWS_PER_TILE

    @pl.when(s < NS - 1)
    def _():
        pltpu.sync_copy(acc_sh.at[pl.ds(s * ROWS_PER_TILE, ROWS_PER_TILE)],
                        out_hbm.at[pl.ds(c * N + s * ROWS_PER_TILE,
                                         ROWS_PER_TILE)])

    @pl.when(s == NS - 1)
    def _():
        pltpu.sync_copy(acc_sh.at[pl.ds((NS - 1) * ROWS_PER_TILE, tail_rows)],
                        out_hbm.at[pl.ds(c * N + (NS - 1) * ROWS_PER_TILE,
                                         tail_rows)])


@functools.cache
def _sc_gather_scale_scatter():
    return pl.kernel(
        _sc_body,
        out_type=jax.ShapeDtypeStruct((NC * N, DIN), jnp.float32),
        mesh=plsc.VectorSubcoreMesh(core_axis_name="c", subcore_axis_name="s",
                                    num_cores=NC, num_subcores=NS),
        scratch_types=[
            pltpu.VMEM((3, 1, K), jnp.int32),
            pltpu.VMEM((3, 1, K), jnp.int32),
            pltpu.VMEM((3, 1, K), jnp.int32),
            pltpu.VMEM((3, 1, K), jnp.int32),
            pltpu.VMEM((3, 1, K), jnp.int32),
            pltpu.VMEM((3, 1, K), jnp.int32),
            pltpu.VMEM((K, DIN), jnp.float32),
            pltpu.VMEM((K, DIN), jnp.float32),
            pltpu.VMEM((K, DIN), jnp.float32),
            pltpu.VMEM_SHARED((NPAD, DIN), jnp.float32),
            pltpu.SemaphoreType.DMA,
            pltpu.SemaphoreType.DMA,
            pltpu.SemaphoreType.DMA,
            pltpu.SemaphoreType.DMA,
            pltpu.SemaphoreType.DMA,
            pltpu.SemaphoreType.DMA,
            pltpu.SemaphoreType.DMA,
            pltpu.SemaphoreType.DMA,
            pltpu.SemaphoreType.DMA,
            pltpu.SemaphoreType.DMA,
        ],
    )


@jax.jit
def kernel(x, edge_index, edge_weight, W):
    src = edge_index[1].reshape(NW, NCH, 1, K)
    dst = edge_index[0].reshape(NW, NCH, 1, K)
    ewb = lax.bitcast_convert_type(edge_weight, jnp.int32).reshape(
        NW, NCH, 1, K)

    partial = _sc_gather_scale_scatter()(x, src, dst, ewb)

    nb = 10
    spec_a = pl.BlockSpec((N // nb, DIN), lambda i: (i, 0))
    spec_b = pl.BlockSpec((N // nb, DIN), lambda i: (i + nb, 0))
    out = pl.pallas_call(
        _proj_body,
        grid=(nb,),
        in_specs=[spec_a, spec_b, pl.BlockSpec((DOUT, DIN), lambda i: (0, 0))],
        out_specs=pl.BlockSpec((N // nb, DOUT), lambda i: (i, 0)),
        out_shape=jax.ShapeDtypeStruct((N, DOUT), jnp.float32),
    )(partial, partial, W)
    return out


# speedup vs baseline: 2.0412x; 1.0053x over previous
import functools

import jax
import jax.numpy as jnp
from jax import lax
from jax.experimental import pallas as pl
from jax.experimental.pallas import tpu as pltpu
from jax.experimental.pallas import tpu_sc as plsc

N = 10000
E = 320000
DIN = 128
DOUT = 128

NC = 2
NS = 16
NW = NC * NS
EP = E // NW
K = 80
NCH = EP // K
EPP = NCH * K
NPAD = 10240
ROWS_PER_TILE = NPAD // NS
FB = DIN // 16


def _proj_body(a_ref, b_ref, w_ref, o_ref):
    o_ref[...] = lax.dot_general(
        a_ref[...] + b_ref[...], w_ref[...], (((1,), (1,)), ((), ())),
        preferred_element_type=jnp.float32)


def _sc_body(x_hbm, src_hbm, dst_hbm, ewb_hbm, out_hbm,
             idx0, idx1, idx2, idx3, idx4, idx5,
             rows0, rows1, rows2, acc_sh,
             si0, si1, si2, si3, si4, si5, sg0, sg1, sg2, ssc):
    c = lax.axis_index("c")
    s = lax.axis_index("s")
    wid = s * NC + c

    idxs = (idx0, idx1, idx2, idx3, idx4, idx5)
    rowss = (rows0, rows1, rows2)
    sis = (si0, si1, si2, si3, si4, si5)
    sgs = (sg0, sg1, sg2)

    def fetch_idx(j, q):
        pltpu.async_copy(src_hbm.at[wid, j], idxs[q].at[0], sis[q])
        pltpu.async_copy(dst_hbm.at[wid, j], idxs[q].at[1], sis[q])
        pltpu.async_copy(ewb_hbm.at[wid, j], idxs[q].at[2], sis[q])

    def wait_idx(q):
        pltpu.make_async_copy(src_hbm.at[wid, 0], idxs[q].at[0], sis[q]).wait()
        pltpu.make_async_copy(dst_hbm.at[wid, 0], idxs[q].at[1], sis[q]).wait()
        pltpu.make_async_copy(ewb_hbm.at[wid, 0], idxs[q].at[2], sis[q]).wait()

    def fire_gather(m, q):
        pltpu.async_copy(x_hbm.at[idxs[q].at[0, 0]], rowss[m], sgs[m])

    def wait_gather(m, q):
        pltpu.make_async_copy(x_hbm.at[idxs[q].at[0, 0]], rowss[m],
                              sgs[m]).wait()

    def scale(m, q):
        rows_v = rowss[m]
        idx_v = idxs[q]

        def grp_body(g, gcarry):
            w16 = lax.bitcast_convert_type(idx_v[2, 0, pl.ds(g * 16, 16)],
                                           jnp.float32)
            base = g * 16
            for e in range(16):
                w = lax.gather(
                    w16, jnp.full((16, 1), e, jnp.int32),
                    lax.GatherDimensionNumbers(
                        offset_dims=(), collapsed_slice_dims=(0,),
                        start_index_map=(0,)),
                    (1,), mode=lax.GatherScatterMode.PROMISE_IN_BOUNDS)
                for f in range(FB):
                    sl = pl.ds(16 * f, 16)
                    rows_v[base + e, sl] = rows_v[base + e, sl] * w
            return gcarry

        lax.fori_loop(0, K // 16, grp_body, 0)

    def fire_scatter(m, q):
        pltpu.async_copy(rowss[m], acc_sh.at[idxs[q].at[1, 0]], ssc,
                         add=True)

    def wait_scatter(m):
        pltpu.make_async_copy(x_hbm.at[pl.ds(0, K)], rowss[m], ssc).wait()

    fetch_idx(0, 0)
    fetch_idx(1, 1)
    fetch_idx(2, 2)
    fetch_idx(3, 3)

    wait_idx(0)
    fire_gather(0, 0)
    wait_idx(1)
    fire_gather(1, 1)

    zero16 = jnp.zeros((16,), jnp.float32)

    def zrow(i, carry):
        for f in range(FB):
            rows2[i, pl.ds(16 * f, 16)] = zero16
        return carry

    lax.fori_loop(0, K, zrow, 0)
    zdescs = [
        pltpu.async_copy(rows2,
                         acc_sh.at[pl.ds(s * ROWS_PER_TILE + r * K, K)],
                         ssc)
        for r in range(ROWS_PER_TILE // K)
    ]
    for d in zdescs:
        d.wait()
    plsc.subcore_barrier()

    def chunk_step(j, m, q, wait_sc, fire_next, do_fetch):
        wait_gather(m, q)
        scale(m, q)
        if wait_sc == "always":
            wait_scatter((m + 2) % 3)
        elif wait_sc == "guard":
            @pl.when(j > 0)
            def _():
                wait_scatter((m + 2) % 3)
        if fire_next:
            wait_idx((q + 2) % 6)
            fire_gather((m + 2) % 3, (q + 2) % 6)
        fire_scatter(m, q)
        if do_fetch:
            fetch_idx(j + 4, (q + 4) % 6)

    def six_body(p, carry):
        j0 = 6 * p
        chunk_step(j0, 0, 0, "guard", True, True)
        chunk_step(j0 + 1, 1, 1, "always", True, True)
        chunk_step(j0 + 2, 2, 2, "always", True, True)
        chunk_step(j0 + 3, 0, 3, "always", True, True)
        chunk_step(j0 + 4, 1, 4, "always", True, True)
        chunk_step(j0 + 5, 2, 5, "always", True, True)
        return carry

    lax.fori_loop(0, NCH // 6, six_body, 0)
    chunk_step(120, 0, 0, "always", True, True)
    chunk_step(121, 1, 1, "always", True, False)
    chunk_step(122, 2, 2, "always", True, False)
    chunk_step(123, 0, 3, "always", False, False)
    chunk_step(124, 1, 4, "always", False, False)
    wait_scatter(1)

    plsc.subcore_barrier()

    tail_rows = N - (NS - 1) * ROWS_PER_TILE

    @pl.when(s < NS - 1)
    def _():
        pltpu.sync_copy(acc_sh.at[pl.ds(s * ROWS_PER_TILE, ROWS_PER_TILE)],
                        out_hbm.at[pl.ds(c * N + s * ROWS_PER_TILE,
                                         ROWS_PER_TILE)])

    @pl.when(s == NS - 1)
    def _():
        pltpu.sync_copy(acc_sh.at[pl.ds((NS - 1) * ROWS_PER_TILE, tail_rows)],
                        out_hbm.at[pl.ds(c * N + (NS - 1) * ROWS_PER_TILE,
                                         tail_rows)])


@functools.cache
def _sc_gather_scale_scatter():
    return pl.kernel(
        _sc_body,
        out_type=jax.ShapeDtypeStruct((NC * N, DIN), jnp.float32),
        mesh=plsc.VectorSubcoreMesh(core_axis_name="c", subcore_axis_name="s",
                                    num_cores=NC, num_subcores=NS),
        scratch_types=[
            pltpu.VMEM((3, 1, K), jnp.int32),
            pltpu.VMEM((3, 1, K), jnp.int32),
            pltpu.VMEM((3, 1, K), jnp.int32),
            pltpu.VMEM((3, 1, K), jnp.int32),
            pltpu.VMEM((3, 1, K), jnp.int32),
            pltpu.VMEM((3, 1, K), jnp.int32),
            pltpu.VMEM((K, DIN), jnp.float32),
            pltpu.VMEM((K, DIN), jnp.float32),
            pltpu.VMEM((K, DIN), jnp.float32),
            pltpu.VMEM_SHARED((NPAD, DIN), jnp.float32),
            pltpu.SemaphoreType.DMA,
            pltpu.SemaphoreType.DMA,
            pltpu.SemaphoreType.DMA,
            pltpu.SemaphoreType.DMA,
            pltpu.SemaphoreType.DMA,
            pltpu.SemaphoreType.DMA,
            pltpu.SemaphoreType.DMA,
            pltpu.SemaphoreType.DMA,
            pltpu.SemaphoreType.DMA,
            pltpu.SemaphoreType.DMA,
        ],
    )


@jax.jit
def kernel(x, edge_index, edge_weight, W):
    src = edge_index[1].reshape(NW, NCH, 1, K)
    dst = edge_index[0].reshape(NW, NCH, 1, K)
    ewb = lax.bitcast_convert_type(edge_weight, jnp.int32).reshape(
        NW, NCH, 1, K)

    partial = _sc_gather_scale_scatter()(x, src, dst, ewb)

    nb = 10
    spec_a = pl.BlockSpec((N // nb, DIN), lambda i: (i, 0))
    spec_b = pl.BlockSpec((N // nb, DIN), lambda i: (i + nb, 0))
    out = pl.pallas_call(
        _proj_body,
        grid=(nb,),
        in_specs=[spec_a, spec_b, pl.BlockSpec((DOUT, DIN), lambda i: (0, 0))],
        out_specs=pl.BlockSpec((N // nb, DOUT), lambda i: (i, 0)),
        out_shape=jax.ShapeDtypeStruct((N, DOUT), jnp.float32),
    )(partial, partial, W)
    return out
